# Initial kernel scaffold; baseline (speedup 1.0000x reference)
import functools

import jax
import jax.numpy as jnp
from jax import lax
from jax.experimental import pallas as pl
from jax.experimental.pallas import tpu as pltpu
from jax.experimental.pallas import tpu_sc as plsc

E = 320000
D = 128
NC, NS, L = 2, 16, 16
NW = NC * NS
EPW = E // NW
B = 80
NCHUNK = EPW // B
UNROLL = 8


def _body(tail_hbm, dst_hbm, typ_hbm, z_hbm, rel_hbm, out_hbm,
          tail_v, dst_v, typ_v, rows_t, rows_r, rows_d, out_v, sem):
    wid = lax.axis_index("s") * NC + lax.axis_index("c")
    base = pl.multiple_of(wid * EPW, 8)

    pltpu.sync_copy(tail_hbm.at[pl.ds(base, EPW)], tail_v)
    pltpu.sync_copy(dst_hbm.at[pl.ds(base, EPW)], dst_v)
    pltpu.sync_copy(typ_hbm.at[pl.ds(base, EPW)], typ_v)

    def chunk_body(c, carry):
        off = pl.multiple_of(c * B, 8)
        cp_t = pltpu.async_copy(z_hbm.at[tail_v.at[pl.ds(off, B)]], rows_t, sem)
        cp_d = pltpu.async_copy(z_hbm.at[dst_v.at[pl.ds(off, B)]], rows_d, sem)
        cp_r = pltpu.async_copy(rel_hbm.at[typ_v.at[pl.ds(off, B)]], rows_r, sem)
        cp_t.wait()
        cp_d.wait()
        cp_r.wait()
        for g in range(B // L):
            e_idx = lax.iota(jnp.int32, (L,)) + g * L

            def d_step(k, acc):
                for u in range(UNROLL):
                    d_idx = jnp.full((L,), k * UNROLL + u, jnp.int32)
                    t = plsc.load_gather(rows_t, [e_idx, d_idx])
                    r = plsc.load_gather(rows_r, [e_idx, d_idx])
                    s = plsc.load_gather(rows_d, [e_idx, d_idx])
                    acc = acc + t * r * s
                return acc

            acc = lax.fori_loop(0, D // UNROLL, d_step,
                                jnp.zeros((L,), jnp.float32))
            out_v[pl.ds(off + g * L, L)] = acc
        return carry

    lax.fori_loop(0, NCHUNK, chunk_body, 0)
    pltpu.sync_copy(out_v, out_hbm.at[pl.ds(base, EPW)])


@jax.jit
def _score(tail, dst, typ, z, rel_emb):
    mesh = plsc.VectorSubcoreMesh(core_axis_name="c", subcore_axis_name="s")
    f = functools.partial(
        pl.kernel,
        mesh=mesh,
        out_type=jax.ShapeDtypeStruct((E,), jnp.float32),
        scratch_types=[
            pltpu.VMEM((EPW,), jnp.int32),
            pltpu.VMEM((EPW,), jnp.int32),
            pltpu.VMEM((EPW,), jnp.int32),
            pltpu.VMEM((B, D), jnp.float32),
            pltpu.VMEM((B, D), jnp.float32),
            pltpu.VMEM((B, D), jnp.float32),
            pltpu.VMEM((EPW,), jnp.float32),
            pltpu.SemaphoreType.DMA,
        ],
    )(_body)
    return f(tail, dst, typ, z, rel_emb)


def kernel(z, edge_index, edge_type, rel_emb):
    tail = edge_index[0].astype(jnp.int32)
    dst = edge_index[1].astype(jnp.int32)
    typ = edge_type.astype(jnp.int32)
    return _score(tail, dst, typ, z, rel_emb)


# speedup vs baseline: 2.4828x; 2.4828x over previous
import functools

import jax
import jax.numpy as jnp
from jax import lax
from jax.experimental import pallas as pl
from jax.experimental.pallas import tpu as pltpu
from jax.experimental.pallas import tpu_sc as plsc

E = 320000
D = 128
NC, NS, L = 2, 16, 16
NW = NC * NS
EPW = E // NW
B = 80
NCHUNK = EPW // B
UNROLL = 8


def _body(tail_hbm, dst_hbm, typ_hbm, z_hbm, rel_hbm, out_hbm,
          tail_v, dst_v, typ_v, rows_t, rows_r, rows_d, out_v, sem):
    wid = lax.axis_index("s") * NC + lax.axis_index("c")
    base = pl.multiple_of(wid * EPW, 8)

    pltpu.sync_copy(tail_hbm.at[pl.ds(base, EPW)], tail_v)
    pltpu.sync_copy(dst_hbm.at[pl.ds(base, EPW)], dst_v)
    pltpu.sync_copy(typ_hbm.at[pl.ds(base, EPW)], typ_v)

    def chunk_body(c, carry):
        off = pl.multiple_of(c * B, 8)
        cp_t = pltpu.async_copy(z_hbm.at[tail_v.at[pl.ds(off, B)]], rows_t, sem)
        cp_d = pltpu.async_copy(z_hbm.at[dst_v.at[pl.ds(off, B)]], rows_d, sem)
        cp_r = pltpu.async_copy(rel_hbm.at[typ_v.at[pl.ds(off, B)]], rows_r, sem)
        cp_t.wait()
        cp_d.wait()
        cp_r.wait()
        lane = lax.iota(jnp.int32, L)

        def g_step(g, carry2):
            vec = jnp.zeros((L,), jnp.float32)
            for j in range(L):
                e = g * L + j
                acc = jnp.zeros((L,), jnp.float32)
                for k in range(D // L):
                    sl = pl.ds(k * L, L)
                    acc = acc + rows_t[e, sl] * rows_r[e, sl] * rows_d[e, sl]
                vec = jnp.where(lane == j, jnp.sum(acc), vec)
            out_v[pl.ds(off + g * L, L)] = vec
            return carry2

        lax.fori_loop(0, B // L, g_step, 0)
        return carry

    lax.fori_loop(0, NCHUNK, chunk_body, 0)
    pltpu.sync_copy(out_v, out_hbm.at[pl.ds(base, EPW)])


@jax.jit
def _score(tail, dst, typ, z, rel_emb):
    mesh = plsc.VectorSubcoreMesh(core_axis_name="c", subcore_axis_name="s")
    f = functools.partial(
        pl.kernel,
        mesh=mesh,
        compiler_params=pltpu.CompilerParams(needs_layout_passes=False),
        out_type=jax.ShapeDtypeStruct((E,), jnp.float32),
        scratch_types=[
            pltpu.VMEM((EPW,), jnp.int32),
            pltpu.VMEM((EPW,), jnp.int32),
            pltpu.VMEM((EPW,), jnp.int32),
            pltpu.VMEM((B, D), jnp.float32),
            pltpu.VMEM((B, D), jnp.float32),
            pltpu.VMEM((B, D), jnp.float32),
            pltpu.VMEM((EPW,), jnp.float32),
            pltpu.SemaphoreType.DMA,
        ],
    )(_body)
    return f(tail, dst, typ, z, rel_emb)


def kernel(z, edge_index, edge_type, rel_emb):
    tail = edge_index[0].astype(jnp.int32)
    dst = edge_index[1].astype(jnp.int32)
    typ = edge_type.astype(jnp.int32)
    return _score(tail, dst, typ, z, rel_emb)


# speedup vs baseline: 3.1983x; 1.2882x over previous
import functools

import jax
import jax.numpy as jnp
from jax import lax
from jax.experimental import pallas as pl
from jax.experimental.pallas import tpu as pltpu
from jax.experimental.pallas import tpu_sc as plsc

E = 320000
D = 128
NC, NS, L = 2, 16, 16
NW = NC * NS
EPW = E // NW
B = 80
NCHUNK = EPW // B
UNROLL = 8


def _body(tail_hbm, dst_hbm, typ_hbm, z_hbm, rel_hbm, out_hbm,
          tail_v, dst_v, typ_v, rows_t, rows_r, rows_d, out_v, sem0, sem1):
    wid = lax.axis_index("s") * NC + lax.axis_index("c")
    base = pl.multiple_of(wid * EPW, 8)
    sems = (sem0, sem1)

    pltpu.sync_copy(tail_hbm.at[pl.ds(base, EPW)], tail_v)
    pltpu.sync_copy(dst_hbm.at[pl.ds(base, EPW)], dst_v)
    pltpu.sync_copy(typ_hbm.at[pl.ds(base, EPW)], typ_v)

    def issue(c, slot):
        off = pl.multiple_of(c * B, 8)
        pltpu.async_copy(z_hbm.at[tail_v.at[pl.ds(off, B)]], rows_t.at[slot],
                         sems[slot])
        pltpu.async_copy(z_hbm.at[dst_v.at[pl.ds(off, B)]], rows_d.at[slot],
                         sems[slot])
        pltpu.async_copy(rel_hbm.at[typ_v.at[pl.ds(off, B)]], rows_r.at[slot],
                         sems[slot])

    def drain(slot):
        for buf in (rows_t, rows_d, rows_r):
            pltpu.make_async_copy(z_hbm.at[pl.ds(0, B)], buf.at[slot],
                                  sems[slot]).wait()

    def compute(c, slot):
        off = pl.multiple_of(c * B, 8)
        rt, rr, rd = rows_t.at[slot], rows_r.at[slot], rows_d.at[slot]
        lane = lax.iota(jnp.int32, L)

        def g_step(g, carry2):
            vec = jnp.zeros((L,), jnp.float32)
            for j in range(L):
                e = g * L + j
                acc = jnp.zeros((L,), jnp.float32)
                for k in range(D // L):
                    sl = pl.ds(k * L, L)
                    acc = acc + rt[e, sl] * rr[e, sl] * rd[e, sl]
                vec = jnp.where(lane == j, jnp.sum(acc), vec)
            out_v[pl.ds(off + g * L, L)] = vec
            return carry2

        lax.fori_loop(0, B // L, g_step, 0)

    issue(0, 0)
    issue(1, 1)

    def pair_body(p, carry):
        c0 = p * 2
        drain(0)
        compute(c0, 0)
        issue(c0 + 2, 0)
        c1 = c0 + 1
        drain(1)
        compute(c1, 1)

        @pl.when(c1 + 2 < NCHUNK)
        def _():
            issue(c1 + 2, 1)

        return carry

    lax.fori_loop(0, (NCHUNK - 1) // 2, pair_body, 0)
    drain(0)
    compute(NCHUNK - 1, 0)
    pltpu.sync_copy(out_v, out_hbm.at[pl.ds(base, EPW)])


@jax.jit
def _score(tail, dst, typ, z, rel_emb):
    mesh = plsc.VectorSubcoreMesh(core_axis_name="c", subcore_axis_name="s")
    f = functools.partial(
        pl.kernel,
        mesh=mesh,
        compiler_params=pltpu.CompilerParams(needs_layout_passes=False),
        out_type=jax.ShapeDtypeStruct((E,), jnp.float32),
        scratch_types=[
            pltpu.VMEM((EPW,), jnp.int32),
            pltpu.VMEM((EPW,), jnp.int32),
            pltpu.VMEM((EPW,), jnp.int32),
            pltpu.VMEM((2, B, D), jnp.float32),
            pltpu.VMEM((2, B, D), jnp.float32),
            pltpu.VMEM((2, B, D), jnp.float32),
            pltpu.VMEM((EPW,), jnp.float32),
            pltpu.SemaphoreType.DMA,
            pltpu.SemaphoreType.DMA,
        ],
    )(_body)
    return f(tail, dst, typ, z, rel_emb)


def kernel(z, edge_index, edge_type, rel_emb):
    tail = edge_index[0].astype(jnp.int32)
    dst = edge_index[1].astype(jnp.int32)
    typ = edge_type.astype(jnp.int32)
    return _score(tail, dst, typ, z, rel_emb)


# speedup vs baseline: 6.0913x; 1.9046x over previous
import functools

import jax
import jax.numpy as jnp
from jax import lax
from jax.experimental import pallas as pl
from jax.experimental.pallas import tpu as pltpu
from jax.experimental.pallas import tpu_sc as plsc

E = 320000
D = 128
N = 10000
R = 1024
NC, NS, L = 2, 16, 16
NW = NC * NS
EPW = E // NW
B = 80
NCHUNK = EPW // B


def _body(tail_hbm, dst_hbm, typ_hbm, z_hbm, rel_hbm, out_hbm,
          z_sp, rel_sp, idx_t, idx_d, idx_r, rows_t, rows_r, rows_d, out_b,
          sem, isem, osem0, osem1):
    wid = lax.axis_index("s") * NC + lax.axis_index("c")
    base = pl.multiple_of(wid * EPW, 8)
    osems = (osem0, osem1)

    @pl.when(lax.axis_index("s") == 0)
    def _():
        pltpu.sync_copy(z_hbm, z_sp)
        pltpu.sync_copy(rel_hbm, rel_sp)

    plsc.subcore_barrier()

    def chunk(c, slot, wait_out):
        off = pl.multiple_of(c * B, 8)
        hoff = pl.multiple_of(base + c * B, 8)
        pltpu.async_copy(tail_hbm.at[pl.ds(hoff, B)], idx_t, isem)
        pltpu.async_copy(dst_hbm.at[pl.ds(hoff, B)], idx_d, isem)
        pltpu.async_copy(typ_hbm.at[pl.ds(hoff, B)], idx_r, isem)
        for _ in range(3):
            pltpu.make_async_copy(tail_hbm.at[pl.ds(0, B)], idx_t, isem).wait()
        pltpu.async_copy(z_sp.at[idx_t], rows_t, sem)
        pltpu.async_copy(z_sp.at[idx_d], rows_d, sem)
        pltpu.async_copy(rel_sp.at[idx_r], rows_r, sem)
        for _ in range(3):
            pltpu.make_async_copy(z_hbm.at[pl.ds(0, B)], rows_t, sem).wait()
        if wait_out:
            pltpu.make_async_copy(out_b.at[slot],
                                  out_hbm.at[pl.ds(0, B)], osems[slot]).wait()

        lane = lax.iota(jnp.int32, L)

        def g_step(g, carry2):
            def e_step(j, vec):
                e = g * L + j
                acc = jnp.zeros((L,), jnp.float32)
                for k in range(D // L):
                    sl = pl.ds(k * L, L)
                    acc = acc + rows_t[e, sl] * rows_r[e, sl] * rows_d[e, sl]
                return jnp.where(lane == j, jnp.sum(acc), vec)

            vec = lax.fori_loop(0, L, e_step, jnp.zeros((L,), jnp.float32))
            out_b[slot, pl.ds(g * L, L)] = vec
            return carry2

        lax.fori_loop(0, B // L, g_step, 0)
        pltpu.async_copy(out_b.at[slot], out_hbm.at[pl.ds(hoff, B)],
                         osems[slot])

    def pair_body(p, carry):
        chunk(p * 2, 0, True)
        chunk(p * 2 + 1, 1, True)
        return carry

    chunk(0, 0, False)
    chunk(1, 1, False)
    lax.fori_loop(1, (NCHUNK - 1) // 2, pair_body, 0)
    chunk(NCHUNK - 1, 0, True)
    pltpu.make_async_copy(out_b.at[1], out_hbm.at[pl.ds(0, B)], osems[1]).wait()
    pltpu.make_async_copy(out_b.at[0], out_hbm.at[pl.ds(0, B)], osems[0]).wait()


@jax.jit
def _score(tail, dst, typ, z, rel_emb):
    mesh = plsc.VectorSubcoreMesh(core_axis_name="c", subcore_axis_name="s")
    f = functools.partial(
        pl.kernel,
        mesh=mesh,
        compiler_params=pltpu.CompilerParams(needs_layout_passes=False),
        out_type=jax.ShapeDtypeStruct((E,), jnp.float32),
        scratch_types=[
            pltpu.VMEM_SHARED((N, D), jnp.float32),
            pltpu.VMEM_SHARED((R, D), jnp.float32),
            pltpu.VMEM((B,), jnp.int32),
            pltpu.VMEM((B,), jnp.int32),
            pltpu.VMEM((B,), jnp.int32),
            pltpu.VMEM((B, D), jnp.float32),
            pltpu.VMEM((B, D), jnp.float32),
            pltpu.VMEM((B, D), jnp.float32),
            pltpu.VMEM((2, B), jnp.float32),
            pltpu.SemaphoreType.DMA,
            pltpu.SemaphoreType.DMA,
            pltpu.SemaphoreType.DMA,
            pltpu.SemaphoreType.DMA,
        ],
    )(_body)
    return f(tail, dst, typ, z, rel_emb)


def kernel(z, edge_index, edge_type, rel_emb):
    tail = edge_index[0].astype(jnp.int32)
    dst = edge_index[1].astype(jnp.int32)
    typ = edge_type.astype(jnp.int32)
    return _score(tail, dst, typ, z, rel_emb)
